# baseline (speedup 1.0000x reference)
import functools

import jax
import jax.numpy as jnp
from jax import lax
from jax.experimental import pallas as pl
from jax.experimental.pallas import tpu as pltpu
from jax.experimental.pallas import tpu_sc as plsc

_ALPHA = 0.95
_LAMBDA = 0.5
_N = 1048576
_K = int(_N * _ALPHA)
_NT = 16
_NE = _N // _NT
_SUB = 8192
_NSUB = _NE // _SUB
_L = 16


def _sc_body(pred_hbm, tgt_hbm, out_hbm,
             err_v, stg_p, stg_t, hist_v, fold_v, gh_v, g_v,
             partial_v, ph_v, out_stage,
             sh1, sh2, sh3, sh4, shf):
    sid = lax.axis_index("s")
    cid = lax.axis_index("c")
    base = sid * _NE
    lane = lax.iota(jnp.int32, _L)
    ones_i = jnp.ones((_L,), jnp.int32)

    def zero_hist(j, c):
        hist_v[pl.ds(j * _L, _L)] = jnp.zeros((_L,), jnp.int32)
        return c

    lax.fori_loop(0, 256, zero_hist, 0)

    def p1_sub(subi, acc):
        off = base + subi * _SUB
        pltpu.sync_copy(pred_hbm.at[pl.ds(off, _SUB)], stg_p)
        pltpu.sync_copy(tgt_hbm.at[pl.ds(off, _SUB)], stg_t)

        def inner(i, a):
            p = stg_p[pl.ds(i * _L, _L)]
            t = stg_t[pl.ds(i * _L, _L)]
            e = jnp.abs(p - t)
            err_v[pl.ds(subi * _SUB + i * _L, _L)] = e
            bits = lax.bitcast_convert_type(e, jnp.int32)
            digit = lax.shift_right_logical(bits, 23)
            plsc.addupdate_scatter(hist_v, [lane * 256 + digit], ones_i)
            return a + e * e

        return lax.fori_loop(0, _SUB // _L, inner, acc)

    acc_sq = lax.fori_loop(0, _NSUB, p1_sub, jnp.zeros((_L,), jnp.float32))
    sum_sq = jnp.sum(acc_sq)

    def fold_publish(sh):
        def fold(j, c):
            acc = hist_v[pl.ds(j * _L, _L)]
            for l in range(1, _NT):
                acc = acc + hist_v[pl.ds(l * 256 + j * _L, _L)]
            fold_v[pl.ds(j * _L, _L)] = acc
            return c

        lax.fori_loop(0, 16, fold, 0)
        pltpu.sync_copy(fold_v, sh.at[sid])
        plsc.subcore_barrier()

    def global_scan(sh, k_rem):
        pltpu.sync_copy(sh, gh_v)

        def foldg(j, c):
            acc = gh_v[0, pl.ds(j * _L, _L)]
            for t in range(1, _NT):
                acc = acc + gh_v[t, pl.ds(j * _L, _L)]
            g_v[pl.ds(j * _L, _L)] = acc
            return c

        lax.fori_loop(0, 16, foldg, 0)

        def sufloop(jj, carry):
            best, run = carry
            j = 15 - jj
            g = g_v[pl.ds(j * _L, _L)]
            tot = jnp.sum(g)
            cum = plsc.cumsum(g)
            s = run + (tot - cum) + g
            ids = j * _L + lane
            cand = jnp.where(s >= k_rem, ids, jnp.int32(-1))
            return jnp.maximum(best, jnp.max(cand)), run + tot

        best, _ = lax.fori_loop(0, 16, sufloop,
                                (jnp.int32(-1), jnp.int32(0)))

        def aboveloop(j, na):
            g = g_v[pl.ds(j * _L, _L)]
            ids = j * _L + lane
            return na + jnp.sum(jnp.where(ids > best, g, jnp.int32(0)))

        n_above = lax.fori_loop(0, 16, aboveloop, jnp.int32(0))
        return best, n_above

    fold_publish(sh1)
    c1, na1 = global_scan(sh1, jnp.int32(_K))
    k_rem = jnp.int32(_K) - na1

    def masked_hist(shift_prev, prefix, shift_cur, maskbits, sh):
        lax.fori_loop(0, 256, zero_hist, 0)

        def inner(i, c):
            e = err_v[pl.ds(i * _L, _L)]
            bits = lax.bitcast_convert_type(e, jnp.int32)
            m = lax.shift_right_logical(bits, shift_prev) == prefix
            digit = lax.shift_right_logical(bits, shift_cur) & maskbits
            plsc.addupdate_scatter(hist_v, [lane * 256 + digit], ones_i,
                                   mask=m)
            return c

        lax.fori_loop(0, _NE // _L, inner, 0)
        fold_publish(sh)

    masked_hist(23, c1, 15, jnp.int32(0xFF), sh2)
    c2, na2 = global_scan(sh2, k_rem)
    k_rem = k_rem - na2
    p2 = (c1 << 8) | c2

    masked_hist(15, p2, 7, jnp.int32(0xFF), sh3)
    c3, na3 = global_scan(sh3, k_rem)
    k_rem = k_rem - na3
    p3 = (p2 << 8) | c3

    masked_hist(7, p3, 0, jnp.int32(0x7F), sh4)
    c4, _na4 = global_scan(sh4, k_rem)
    t_bits = (p3 << 7) | c4

    def finloop(i, carry):
        cnt, s = carry
        e = err_v[pl.ds(i * _L, _L)]
        bits = lax.bitcast_convert_type(e, jnp.int32)
        gt = bits > t_bits
        cnt = cnt + jnp.where(gt, jnp.int32(1), jnp.int32(0))
        s = s + jnp.where(gt, e, jnp.float32(0.0))
        return cnt, s

    cntv, sv = lax.fori_loop(
        0, _NE // _L, finloop,
        (jnp.zeros((_L,), jnp.int32), jnp.zeros((_L,), jnp.float32)))
    cnt_gt = jnp.sum(cntv).astype(jnp.float32)
    s_gt = jnp.sum(sv)

    pv = jnp.where(lane == 0, sum_sq,
                   jnp.where(lane == 1, s_gt,
                             jnp.where(lane == 2, cnt_gt, jnp.float32(0.0))))
    def zero_partial(j, c):
        partial_v[pl.ds(j * _L, _L)] = jnp.zeros((_L,), jnp.float32)
        return c
    lax.fori_loop(0, 256 // _L, zero_partial, 0)
    partial_v[pl.ds(0, _L)] = pv
    pltpu.sync_copy(partial_v, shf.at[sid])
    plsc.subcore_barrier()

    @pl.when(jnp.logical_and(sid == 0, cid == 0))
    def _():
        pltpu.sync_copy(shf, ph_v)
        acc = ph_v[0, pl.ds(0, _L)]
        for t in range(1, _NT):
            acc = acc + ph_v[t, pl.ds(0, _L)]
        tot_sumsq = jnp.sum(jnp.where(lane == 0, acc, jnp.float32(0.0)))
        tot_sgt = jnp.sum(jnp.where(lane == 1, acc, jnp.float32(0.0)))
        tot_cnt = jnp.sum(jnp.where(lane == 2, acc, jnp.float32(0.0)))
        t_val = jnp.max(lax.bitcast_convert_type(
            jnp.full((_L,), t_bits, jnp.int32), jnp.float32))
        kf = jnp.float32(_K)
        cvar = (tot_sgt + (kf - tot_cnt) * t_val) * jnp.float32(1.0 / _K)
        mse = tot_sumsq * jnp.float32(1.0 / _N)
        res = (1.0 - _LAMBDA) * mse + _LAMBDA * cvar
        out_stage[0, pl.ds(0, _L)] = jnp.full((_L,), res, jnp.float32)
        pltpu.sync_copy(out_stage, out_hbm)


_sc_call = functools.partial(
    pl.kernel,
    mesh=plsc.VectorSubcoreMesh(core_axis_name="c", subcore_axis_name="s"),
    out_type=jax.ShapeDtypeStruct((1, _L), jnp.float32),
    compiler_params=pltpu.CompilerParams(needs_layout_passes=False),
    scratch_types=[
        pltpu.VMEM((_NE,), jnp.float32),
        pltpu.VMEM((_SUB,), jnp.float32),
        pltpu.VMEM((_SUB,), jnp.float32),
        pltpu.VMEM((_NT * 256,), jnp.int32),
        pltpu.VMEM((256,), jnp.int32),
        pltpu.VMEM((_NT, 256), jnp.int32),
        pltpu.VMEM((256,), jnp.int32),
        pltpu.VMEM((256,), jnp.float32),
        pltpu.VMEM((_NT, 256), jnp.float32),
        pltpu.VMEM((1, _L), jnp.float32),
        pltpu.VMEM_SHARED((_NT, 256), jnp.int32),
        pltpu.VMEM_SHARED((_NT, 256), jnp.int32),
        pltpu.VMEM_SHARED((_NT, 256), jnp.int32),
        pltpu.VMEM_SHARED((_NT, 256), jnp.int32),
        pltpu.VMEM_SHARED((_NT, 256), jnp.float32),
    ],
)(_sc_body)


@jax.jit
def kernel(pred, target):
    out = _sc_call(pred, target)
    return out[0, 0]


# speedup vs baseline: 1.1976x; 1.1976x over previous
import functools

import jax
import jax.numpy as jnp
from jax import lax
from jax.experimental import pallas as pl
from jax.experimental.pallas import tpu as pltpu
from jax.experimental.pallas import tpu_sc as plsc

_ALPHA = 0.95
_LAMBDA = 0.5
_N = 1048576
_K = int(_N * _ALPHA)
_NT = 16
_NE = _N // _NT
_SUB = 8192
_NSUB = _NE // _SUB
_L = 16


def _sc_body(pred_hbm, tgt_hbm, out_hbm,
             err_v, stg_p, stg_t, hist_v, fold_v, gh_v, g_v,
             partial_v, ph_v, out_stage,
             sh1, sh2, sh3, sh4, shf):
    sid = lax.axis_index("s")
    cid = lax.axis_index("c")
    base = sid * _NE
    lane = lax.iota(jnp.int32, _L)
    ones_i = jnp.ones((_L,), jnp.int32)

    def zero_hist(j, c):
        hist_v[pl.ds(j * _L, _L)] = jnp.zeros((_L,), jnp.int32)
        return c

    lax.fori_loop(0, 257, zero_hist, 0, unroll=8)

    def p1_sub(subi, acc):
        off = base + subi * _SUB
        pltpu.sync_copy(pred_hbm.at[pl.ds(off, _SUB)], stg_p)
        pltpu.sync_copy(tgt_hbm.at[pl.ds(off, _SUB)], stg_t)

        def inner(i, a):
            p = stg_p[pl.ds(i * _L, _L)]
            t = stg_t[pl.ds(i * _L, _L)]
            e = jnp.abs(p - t)
            err_v[pl.ds(subi * _SUB + i * _L, _L)] = e
            bits = lax.bitcast_convert_type(e, jnp.int32)
            digit = lax.shift_right_logical(bits, 23)
            plsc.addupdate_scatter(hist_v, [lane * 257 + digit], ones_i)
            return a + e * e

        return lax.fori_loop(0, _SUB // _L, inner, acc, unroll=8)

    acc_sq = lax.fori_loop(0, _NSUB, p1_sub, jnp.zeros((_L,), jnp.float32))
    sum_sq = jnp.sum(acc_sq)

    def fold_publish(sh):
        def fold(j, c):
            acc = hist_v[pl.ds(j * _L, _L)]
            for l in range(1, _NT):
                acc = acc + hist_v[pl.ds(l * 257 + j * _L, _L)]
            fold_v[pl.ds(j * _L, _L)] = acc
            return c

        lax.fori_loop(0, 16, fold, 0)
        pltpu.sync_copy(fold_v, sh.at[sid])
        plsc.subcore_barrier()

    def global_scan(sh, k_rem):
        pltpu.sync_copy(sh, gh_v)

        def foldg(j, c):
            acc = gh_v[0, pl.ds(j * _L, _L)]
            for t in range(1, _NT):
                acc = acc + gh_v[t, pl.ds(j * _L, _L)]
            g_v[pl.ds(j * _L, _L)] = acc
            return c

        lax.fori_loop(0, 16, foldg, 0)

        def sufloop(jj, carry):
            best, run = carry
            j = 15 - jj
            g = g_v[pl.ds(j * _L, _L)]
            tot = jnp.sum(g)
            cum = plsc.cumsum(g)
            s = run + (tot - cum) + g
            ids = j * _L + lane
            cand = jnp.where(s >= k_rem, ids, jnp.int32(-1))
            return jnp.maximum(best, jnp.max(cand)), run + tot

        best, _ = lax.fori_loop(0, 16, sufloop,
                                (jnp.int32(-1), jnp.int32(0)))

        def aboveloop(j, na):
            g = g_v[pl.ds(j * _L, _L)]
            ids = j * _L + lane
            return na + jnp.sum(jnp.where(ids > best, g, jnp.int32(0)))

        n_above = lax.fori_loop(0, 16, aboveloop, jnp.int32(0))
        return best, n_above

    fold_publish(sh1)
    c1, na1 = global_scan(sh1, jnp.int32(_K))
    k_rem = jnp.int32(_K) - na1

    def masked_hist(shift_prev, prefix, shift_cur, maskbits, sh):
        lax.fori_loop(0, 257, zero_hist, 0, unroll=8)

        def inner(i, c):
            e = err_v[pl.ds(i * _L, _L)]
            bits = lax.bitcast_convert_type(e, jnp.int32)
            m = lax.shift_right_logical(bits, shift_prev) == prefix
            digit = lax.shift_right_logical(bits, shift_cur) & maskbits
            plsc.addupdate_scatter(hist_v, [lane * 257 + digit], ones_i,
                                   mask=m)
            return c

        lax.fori_loop(0, _NE // _L, inner, 0, unroll=8)
        fold_publish(sh)

    masked_hist(23, c1, 15, jnp.int32(0xFF), sh2)
    c2, na2 = global_scan(sh2, k_rem)
    k_rem = k_rem - na2
    p2 = (c1 << 8) | c2

    masked_hist(15, p2, 7, jnp.int32(0xFF), sh3)
    c3, na3 = global_scan(sh3, k_rem)
    k_rem = k_rem - na3
    p3 = (p2 << 8) | c3

    masked_hist(7, p3, 0, jnp.int32(0x7F), sh4)
    c4, _na4 = global_scan(sh4, k_rem)
    t_bits = (p3 << 7) | c4

    def finloop(i, carry):
        cnt, s = carry
        e = err_v[pl.ds(i * _L, _L)]
        bits = lax.bitcast_convert_type(e, jnp.int32)
        gt = bits > t_bits
        cnt = cnt + jnp.where(gt, jnp.int32(1), jnp.int32(0))
        s = s + jnp.where(gt, e, jnp.float32(0.0))
        return cnt, s

    cntv, sv = lax.fori_loop(
        0, _NE // _L, finloop,
        (jnp.zeros((_L,), jnp.int32), jnp.zeros((_L,), jnp.float32)),
        unroll=8)
    cnt_gt = jnp.sum(cntv).astype(jnp.float32)
    s_gt = jnp.sum(sv)

    pv = jnp.where(lane == 0, sum_sq,
                   jnp.where(lane == 1, s_gt,
                             jnp.where(lane == 2, cnt_gt, jnp.float32(0.0))))
    def zero_partial(j, c):
        partial_v[pl.ds(j * _L, _L)] = jnp.zeros((_L,), jnp.float32)
        return c
    lax.fori_loop(0, 256 // _L, zero_partial, 0)
    partial_v[pl.ds(0, _L)] = pv
    pltpu.sync_copy(partial_v, shf.at[sid])
    plsc.subcore_barrier()

    @pl.when(jnp.logical_and(sid == 0, cid == 0))
    def _():
        pltpu.sync_copy(shf, ph_v)
        acc = ph_v[0, pl.ds(0, _L)]
        for t in range(1, _NT):
            acc = acc + ph_v[t, pl.ds(0, _L)]
        tot_sumsq = jnp.sum(jnp.where(lane == 0, acc, jnp.float32(0.0)))
        tot_sgt = jnp.sum(jnp.where(lane == 1, acc, jnp.float32(0.0)))
        tot_cnt = jnp.sum(jnp.where(lane == 2, acc, jnp.float32(0.0)))
        t_val = jnp.max(lax.bitcast_convert_type(
            jnp.full((_L,), t_bits, jnp.int32), jnp.float32))
        kf = jnp.float32(_K)
        cvar = (tot_sgt + (kf - tot_cnt) * t_val) * jnp.float32(1.0 / _K)
        mse = tot_sumsq * jnp.float32(1.0 / _N)
        res = (1.0 - _LAMBDA) * mse + _LAMBDA * cvar
        out_stage[0, pl.ds(0, _L)] = jnp.full((_L,), res, jnp.float32)
        pltpu.sync_copy(out_stage, out_hbm)


_sc_call = functools.partial(
    pl.kernel,
    mesh=plsc.VectorSubcoreMesh(core_axis_name="c", subcore_axis_name="s"),
    out_type=jax.ShapeDtypeStruct((1, _L), jnp.float32),
    compiler_params=pltpu.CompilerParams(needs_layout_passes=False),
    scratch_types=[
        pltpu.VMEM((_NE,), jnp.float32),
        pltpu.VMEM((_SUB,), jnp.float32),
        pltpu.VMEM((_SUB,), jnp.float32),
        pltpu.VMEM((_NT * 257,), jnp.int32),
        pltpu.VMEM((256,), jnp.int32),
        pltpu.VMEM((_NT, 256), jnp.int32),
        pltpu.VMEM((256,), jnp.int32),
        pltpu.VMEM((256,), jnp.float32),
        pltpu.VMEM((_NT, 256), jnp.float32),
        pltpu.VMEM((1, _L), jnp.float32),
        pltpu.VMEM_SHARED((_NT, 256), jnp.int32),
        pltpu.VMEM_SHARED((_NT, 256), jnp.int32),
        pltpu.VMEM_SHARED((_NT, 256), jnp.int32),
        pltpu.VMEM_SHARED((_NT, 256), jnp.int32),
        pltpu.VMEM_SHARED((_NT, 256), jnp.float32),
    ],
)(_sc_body)


@jax.jit
def kernel(pred, target):
    out = _sc_call(pred, target)
    return out[0, 0]


# speedup vs baseline: 1.3821x; 1.1540x over previous
import functools

import jax
import jax.numpy as jnp
from jax import lax
from jax.experimental import pallas as pl
from jax.experimental.pallas import tpu as pltpu
from jax.experimental.pallas import tpu_sc as plsc

_ALPHA = 0.95
_LAMBDA = 0.5
_N = 1048576
_K = int(_N * _ALPHA)
_NT = 16
_NE = _N // _NT
_SUB = 8192
_NSUB = _NE // _SUB
_L = 16
_HS = 257


def _sc_body(pred_hbm, tgt_hbm, out_hbm,
             err_v, stg_p, stg_t, hist_v, vhist_v, fold_v, vfold_v,
             gh_v, vgh_v, g_v, gv_v, partial_v, ph_v, out_stage,
             sh1, sh2, sh3, sh4, vsh1, vsh2, vsh3, vsh4, shf):
    sid = lax.axis_index("s")
    cid = lax.axis_index("c")
    base = sid * _NE
    lane = lax.iota(jnp.int32, _L)
    ones_i = jnp.ones((_L,), jnp.int32)

    def zero_hists(j, c):
        hist_v[pl.ds(j * _L, _L)] = jnp.zeros((_L,), jnp.int32)
        vhist_v[pl.ds(j * _L, _L)] = jnp.zeros((_L,), jnp.float32)
        return c

    lax.fori_loop(0, _NT * _HS // _L, zero_hists, 0, unroll=8)

    def p1_sub(subi, acc):
        off = base + subi * _SUB
        pltpu.sync_copy(pred_hbm.at[pl.ds(off, _SUB)], stg_p)
        pltpu.sync_copy(tgt_hbm.at[pl.ds(off, _SUB)], stg_t)

        def inner(i, a):
            p = stg_p[pl.ds(i * _L, _L)]
            t = stg_t[pl.ds(i * _L, _L)]
            e = jnp.abs(p - t)
            err_v[pl.ds(subi * _SUB + i * _L, _L)] = e
            bits = lax.bitcast_convert_type(e, jnp.int32)
            idx = lane * _HS + lax.shift_right_logical(bits, 23)
            plsc.addupdate_scatter(hist_v, [idx], ones_i)
            plsc.addupdate_scatter(vhist_v, [idx], e)
            return a + e * e

        return lax.fori_loop(0, _SUB // _L, inner, acc, unroll=8)

    acc_sq = lax.fori_loop(0, _NSUB, p1_sub, jnp.zeros((_L,), jnp.float32))
    sum_sq = jnp.sum(acc_sq)

    def fold_publish(sh, vsh):
        def fold(j, c):
            acc = hist_v[pl.ds(j * _L, _L)]
            vacc = vhist_v[pl.ds(j * _L, _L)]
            for l in range(1, _NT):
                acc = acc + hist_v[pl.ds(l * _HS + j * _L, _L)]
                vacc = vacc + vhist_v[pl.ds(l * _HS + j * _L, _L)]
            fold_v[pl.ds(j * _L, _L)] = acc
            vfold_v[pl.ds(j * _L, _L)] = vacc
            return c

        lax.fori_loop(0, 16, fold, 0)
        pltpu.sync_copy(fold_v, sh.at[sid])
        pltpu.sync_copy(vfold_v, vsh.at[sid])
        plsc.subcore_barrier()

    def global_scan(sh, vsh, k_rem):
        pltpu.sync_copy(sh, gh_v)
        pltpu.sync_copy(vsh, vgh_v)

        def foldg(j, c):
            acc = gh_v[0, pl.ds(j * _L, _L)]
            vacc = vgh_v[0, pl.ds(j * _L, _L)]
            for t in range(1, _NT):
                acc = acc + gh_v[t, pl.ds(j * _L, _L)]
                vacc = vacc + vgh_v[t, pl.ds(j * _L, _L)]
            g_v[pl.ds(j * _L, _L)] = acc
            gv_v[pl.ds(j * _L, _L)] = vacc
            return c

        lax.fori_loop(0, 16, foldg, 0)

        def sufloop(jj, carry):
            best, run = carry
            j = 15 - jj
            g = g_v[pl.ds(j * _L, _L)]
            tot = jnp.sum(g)
            cum = plsc.cumsum(g)
            s = run + (tot - cum) + g
            ids = j * _L + lane
            cand = jnp.where(s >= k_rem, ids, jnp.int32(-1))
            return jnp.maximum(best, jnp.max(cand)), run + tot

        best, _ = lax.fori_loop(0, 16, sufloop,
                                (jnp.int32(-1), jnp.int32(0)))

        def aboveloop(j, carry):
            na, va = carry
            g = g_v[pl.ds(j * _L, _L)]
            gv = gv_v[pl.ds(j * _L, _L)]
            ids = j * _L + lane
            gtm = ids > best
            na = na + jnp.sum(jnp.where(gtm, g, jnp.int32(0)))
            va = va + jnp.sum(jnp.where(gtm, gv, jnp.float32(0.0)))
            return na, va

        n_above, v_above = lax.fori_loop(
            0, 16, aboveloop, (jnp.int32(0), jnp.float32(0.0)))
        return best, n_above, v_above

    fold_publish(sh1, vsh1)
    c1, na1, va1 = global_scan(sh1, vsh1, jnp.int32(_K))
    k_rem = jnp.int32(_K) - na1

    lax.fori_loop(0, _NT * _HS // _L, zero_hists, 0, unroll=8)

    def l2loop(i, off):
        e = err_v[pl.ds(i * _L, _L)]
        bits = lax.bitcast_convert_type(e, jnp.int32)
        m = lax.shift_right_logical(bits, 23) == c1
        idx = lane * _HS + (lax.shift_right_logical(bits, 15) & 0xFF)
        plsc.addupdate_scatter(hist_v, [idx], ones_i, mask=m)
        plsc.addupdate_scatter(vhist_v, [idx], e, mask=m)
        mcnt = plsc.cumsum(jnp.where(m, jnp.int32(1), jnp.int32(0)))
        plsc.store_scatter(err_v, [off + mcnt - 1], e, mask=m)
        return off + jnp.max(mcnt)

    n2 = lax.fori_loop(0, _NE // _L, l2loop, jnp.int32(0), unroll=8)
    fold_publish(sh2, vsh2)
    c2, na2, va2 = global_scan(sh2, vsh2, k_rem)
    k_rem = k_rem - na2
    p2 = (c1 << 8) | c2

    lax.fori_loop(0, _NT * _HS // _L, zero_hists, 0, unroll=8)

    def l3loop(i, off):
        e = err_v[pl.ds(i * _L, _L)]
        bits = lax.bitcast_convert_type(e, jnp.int32)
        valid = (i * _L + lane) < n2
        m = jnp.logical_and(lax.shift_right_logical(bits, 15) == p2, valid)
        idx = lane * _HS + (lax.shift_right_logical(bits, 7) & 0xFF)
        plsc.addupdate_scatter(hist_v, [idx], ones_i, mask=m)
        plsc.addupdate_scatter(vhist_v, [idx], e, mask=m)
        mcnt = plsc.cumsum(jnp.where(m, jnp.int32(1), jnp.int32(0)))
        plsc.store_scatter(err_v, [off + mcnt - 1], e, mask=m)
        return off + jnp.max(mcnt)

    n3 = lax.fori_loop(0, lax.shift_right_logical(n2 + _L - 1, 4), l3loop, jnp.int32(0))
    fold_publish(sh3, vsh3)
    c3, na3, va3 = global_scan(sh3, vsh3, k_rem)
    k_rem = k_rem - na3
    p3 = (p2 << 8) | c3

    lax.fori_loop(0, _NT * _HS // _L, zero_hists, 0, unroll=8)

    def l4loop(i, c):
        e = err_v[pl.ds(i * _L, _L)]
        bits = lax.bitcast_convert_type(e, jnp.int32)
        valid = (i * _L + lane) < n3
        m = jnp.logical_and(lax.shift_right_logical(bits, 7) == p3, valid)
        idx = lane * _HS + (bits & 0x7F)
        plsc.addupdate_scatter(hist_v, [idx], ones_i, mask=m)
        plsc.addupdate_scatter(vhist_v, [idx], e, mask=m)
        return c

    lax.fori_loop(0, lax.shift_right_logical(n3 + _L - 1, 4), l4loop, 0)
    fold_publish(sh4, vsh4)
    c4, na4, va4 = global_scan(sh4, vsh4, k_rem)
    t_bits = (p3 << 7) | c4

    cnt_gt = (na1 + na2 + na3 + na4).astype(jnp.float32)
    s_gt = va1 + va2 + va3 + va4

    def zero_partial(j, c):
        partial_v[pl.ds(j * _L, _L)] = jnp.zeros((_L,), jnp.float32)
        return c

    lax.fori_loop(0, 256 // _L, zero_partial, 0)
    partial_v[pl.ds(0, _L)] = jnp.where(lane == 0, sum_sq, jnp.float32(0.0))
    pltpu.sync_copy(partial_v, shf.at[sid])
    plsc.subcore_barrier()

    @pl.when(jnp.logical_and(sid == 0, cid == 0))
    def _():
        pltpu.sync_copy(shf, ph_v)
        acc = ph_v[0, pl.ds(0, _L)]
        for t in range(1, _NT):
            acc = acc + ph_v[t, pl.ds(0, _L)]
        tot_sumsq = jnp.sum(jnp.where(lane == 0, acc, jnp.float32(0.0)))
        t_val = jnp.max(lax.bitcast_convert_type(
            jnp.full((_L,), t_bits, jnp.int32), jnp.float32))
        kf = jnp.float32(_K)
        cvar = (s_gt + (kf - cnt_gt) * t_val) * jnp.float32(1.0 / _K)
        mse = tot_sumsq * jnp.float32(1.0 / _N)
        res = (1.0 - _LAMBDA) * mse + _LAMBDA * cvar
        out_stage[0, pl.ds(0, _L)] = jnp.full((_L,), res, jnp.float32)
        pltpu.sync_copy(out_stage, out_hbm)


_sc_call = functools.partial(
    pl.kernel,
    mesh=plsc.VectorSubcoreMesh(core_axis_name="c", subcore_axis_name="s"),
    out_type=jax.ShapeDtypeStruct((1, _L), jnp.float32),
    compiler_params=pltpu.CompilerParams(needs_layout_passes=False),
    scratch_types=[
        pltpu.VMEM((_NE,), jnp.float32),
        pltpu.VMEM((_SUB,), jnp.float32),
        pltpu.VMEM((_SUB,), jnp.float32),
        pltpu.VMEM((_NT * _HS,), jnp.int32),
        pltpu.VMEM((_NT * _HS,), jnp.float32),
        pltpu.VMEM((256,), jnp.int32),
        pltpu.VMEM((256,), jnp.float32),
        pltpu.VMEM((_NT, 256), jnp.int32),
        pltpu.VMEM((_NT, 256), jnp.float32),
        pltpu.VMEM((256,), jnp.int32),
        pltpu.VMEM((256,), jnp.float32),
        pltpu.VMEM((256,), jnp.float32),
        pltpu.VMEM((_NT, 256), jnp.float32),
        pltpu.VMEM((1, _L), jnp.float32),
        pltpu.VMEM_SHARED((_NT, 256), jnp.int32),
        pltpu.VMEM_SHARED((_NT, 256), jnp.int32),
        pltpu.VMEM_SHARED((_NT, 256), jnp.int32),
        pltpu.VMEM_SHARED((_NT, 256), jnp.int32),
        pltpu.VMEM_SHARED((_NT, 256), jnp.float32),
        pltpu.VMEM_SHARED((_NT, 256), jnp.float32),
        pltpu.VMEM_SHARED((_NT, 256), jnp.float32),
        pltpu.VMEM_SHARED((_NT, 256), jnp.float32),
        pltpu.VMEM_SHARED((_NT, 256), jnp.float32),
    ],
)(_sc_body)


@jax.jit
def kernel(pred, target):
    out = _sc_call(pred, target)
    return out[0, 0]


# speedup vs baseline: 1.4697x; 1.0634x over previous
import functools

import jax
import jax.numpy as jnp
from jax import lax
from jax.experimental import pallas as pl
from jax.experimental.pallas import tpu as pltpu
from jax.experimental.pallas import tpu_sc as plsc

_ALPHA = 0.95
_LAMBDA = 0.5
_N = 1048576
_K = int(_N * _ALPHA)
_NT = 16
_NE = _N // _NT
_SUB = 8192
_NSUB = _NE // _SUB
_L = 16
_HS = 257


def _sc_body(pred_hbm, tgt_hbm, out_hbm,
             err_v, stg_p, stg_t, hist_v, vhist_v, fold_v, vfold_v,
             gh_v, vgh_v, g_v, gv_v, partial_v, ph_v, out_stage,
             sh1, sh2, sh3, sh4, vsh1, vsh2, vsh3, vsh4, shf):
    sid = lax.axis_index("s")
    cid = lax.axis_index("c")
    base = sid * _NE
    lane = lax.iota(jnp.int32, _L)
    ones_i = jnp.ones((_L,), jnp.int32)

    def zero_hists(j, c):
        hist_v[pl.ds(j * _L, _L)] = jnp.zeros((_L,), jnp.int32)
        vhist_v[pl.ds(j * _L, _L)] = jnp.zeros((_L,), jnp.float32)
        return c

    lax.fori_loop(0, _NT * _HS // _L, zero_hists, 0, unroll=8)

    def p1_sub(subi, acc):
        off = base + subi * _SUB
        pltpu.sync_copy(pred_hbm.at[pl.ds(off, _SUB)], stg_p)
        pltpu.sync_copy(tgt_hbm.at[pl.ds(off, _SUB)], stg_t)

        def inner(i, a):
            p = stg_p[pl.ds(i * _L, _L)]
            t = stg_t[pl.ds(i * _L, _L)]
            e = jnp.abs(p - t)
            err_v[pl.ds(subi * _SUB + i * _L, _L)] = e
            bits = lax.bitcast_convert_type(e, jnp.int32)
            idx = lane * _HS + lax.shift_right_logical(bits, 23)
            plsc.addupdate_scatter(hist_v, [idx], ones_i)
            plsc.addupdate_scatter(vhist_v, [idx], e)
            return a + e * e

        return lax.fori_loop(0, _SUB // _L, inner, acc, unroll=8)

    acc_sq = lax.fori_loop(0, _NSUB, p1_sub, jnp.zeros((_L,), jnp.float32))
    sum_sq = jnp.sum(acc_sq)

    def fold_publish(sh, vsh):
        def fold(j, c):
            acc = hist_v[pl.ds(j * _L, _L)]
            vacc = vhist_v[pl.ds(j * _L, _L)]
            for l in range(1, _NT):
                acc = acc + hist_v[pl.ds(l * _HS + j * _L, _L)]
                vacc = vacc + vhist_v[pl.ds(l * _HS + j * _L, _L)]
            fold_v[pl.ds(j * _L, _L)] = acc
            vfold_v[pl.ds(j * _L, _L)] = vacc
            return c

        lax.fori_loop(0, 16, fold, 0)
        pltpu.sync_copy(fold_v, sh.at[sid])
        pltpu.sync_copy(vfold_v, vsh.at[sid])
        plsc.subcore_barrier()

    def global_scan(sh, vsh, k_rem):
        pltpu.sync_copy(sh, gh_v)
        pltpu.sync_copy(vsh, vgh_v)

        def foldg(j, c):
            acc = gh_v[0, pl.ds(j * _L, _L)]
            vacc = vgh_v[0, pl.ds(j * _L, _L)]
            for t in range(1, _NT):
                acc = acc + gh_v[t, pl.ds(j * _L, _L)]
                vacc = vacc + vgh_v[t, pl.ds(j * _L, _L)]
            g_v[pl.ds(j * _L, _L)] = acc
            gv_v[pl.ds(j * _L, _L)] = vacc
            return c

        lax.fori_loop(0, 16, foldg, 0)

        def sufloop(jj, carry):
            best, run = carry
            j = 15 - jj
            g = g_v[pl.ds(j * _L, _L)]
            tot = jnp.sum(g)
            cum = plsc.cumsum(g)
            s = run + (tot - cum) + g
            ids = j * _L + lane
            cand = jnp.where(s >= k_rem, ids, jnp.int32(-1))
            return jnp.maximum(best, jnp.max(cand)), run + tot

        best, _ = lax.fori_loop(0, 16, sufloop,
                                (jnp.int32(-1), jnp.int32(0)))

        def aboveloop(j, carry):
            na, va = carry
            g = g_v[pl.ds(j * _L, _L)]
            gv = gv_v[pl.ds(j * _L, _L)]
            ids = j * _L + lane
            gtm = ids > best
            na = na + jnp.sum(jnp.where(gtm, g, jnp.int32(0)))
            va = va + jnp.sum(jnp.where(gtm, gv, jnp.float32(0.0)))
            return na, va

        n_above, v_above = lax.fori_loop(
            0, 16, aboveloop, (jnp.int32(0), jnp.float32(0.0)))
        return best, n_above, v_above

    fold_publish(sh1, vsh1)
    c1, na1, va1 = global_scan(sh1, vsh1, jnp.int32(_K))
    k_rem = jnp.int32(_K) - na1

    lax.fori_loop(0, _NT * _HS // _L, zero_hists, 0, unroll=8)

    def l2loop(i, off):
        e = err_v[pl.ds(i * _L, _L)]
        bits = lax.bitcast_convert_type(e, jnp.int32)
        m = lax.shift_right_logical(bits, 23) == c1
        idx = lane * _HS + (lax.shift_right_logical(bits, 15) & 0xFF)
        plsc.addupdate_scatter(hist_v, [idx], ones_i, mask=m)
        plsc.addupdate_scatter(vhist_v, [idx], e, mask=m)
        mcnt = plsc.cumsum(jnp.where(m, jnp.int32(1), jnp.int32(0)))
        plsc.store_scatter(err_v, [off + mcnt - 1], e, mask=m)
        return off + plsc.all_reduce_population_count(m)

    off2 = lax.fori_loop(0, _NE // _L, l2loop,
                         jnp.zeros((_L,), jnp.int32), unroll=8)
    n2 = jnp.max(off2)
    fold_publish(sh2, vsh2)
    c2, na2, va2 = global_scan(sh2, vsh2, k_rem)
    k_rem = k_rem - na2
    p2 = (c1 << 8) | c2

    lax.fori_loop(0, _NT * _HS // _L, zero_hists, 0, unroll=8)

    def l3loop(i, off):
        e = err_v[pl.ds(i * _L, _L)]
        bits = lax.bitcast_convert_type(e, jnp.int32)
        valid = (i * _L + lane) < n2
        m = jnp.logical_and(lax.shift_right_logical(bits, 15) == p2, valid)
        idx = lane * _HS + (lax.shift_right_logical(bits, 7) & 0xFF)
        plsc.addupdate_scatter(hist_v, [idx], ones_i, mask=m)
        plsc.addupdate_scatter(vhist_v, [idx], e, mask=m)
        mcnt = plsc.cumsum(jnp.where(m, jnp.int32(1), jnp.int32(0)))
        plsc.store_scatter(err_v, [off + mcnt - 1], e, mask=m)
        return off + plsc.all_reduce_population_count(m)

    off3 = lax.fori_loop(0, lax.shift_right_logical(n2 + _L - 1, 4), l3loop,
                         jnp.zeros((_L,), jnp.int32))
    n3 = jnp.max(off3)
    fold_publish(sh3, vsh3)
    c3, na3, va3 = global_scan(sh3, vsh3, k_rem)
    k_rem = k_rem - na3
    p3 = (p2 << 8) | c3

    lax.fori_loop(0, _NT * _HS // _L, zero_hists, 0, unroll=8)

    def l4loop(i, c):
        e = err_v[pl.ds(i * _L, _L)]
        bits = lax.bitcast_convert_type(e, jnp.int32)
        valid = (i * _L + lane) < n3
        m = jnp.logical_and(lax.shift_right_logical(bits, 7) == p3, valid)
        idx = lane * _HS + (bits & 0x7F)
        plsc.addupdate_scatter(hist_v, [idx], ones_i, mask=m)
        plsc.addupdate_scatter(vhist_v, [idx], e, mask=m)
        return c

    lax.fori_loop(0, lax.shift_right_logical(n3 + _L - 1, 4), l4loop, 0)
    fold_publish(sh4, vsh4)
    c4, na4, va4 = global_scan(sh4, vsh4, k_rem)
    t_bits = (p3 << 7) | c4

    cnt_gt = (na1 + na2 + na3 + na4).astype(jnp.float32)
    s_gt = va1 + va2 + va3 + va4

    def zero_partial(j, c):
        partial_v[pl.ds(j * _L, _L)] = jnp.zeros((_L,), jnp.float32)
        return c

    lax.fori_loop(0, 256 // _L, zero_partial, 0)
    partial_v[pl.ds(0, _L)] = jnp.where(lane == 0, sum_sq, jnp.float32(0.0))
    pltpu.sync_copy(partial_v, shf.at[sid])
    plsc.subcore_barrier()

    @pl.when(jnp.logical_and(sid == 0, cid == 0))
    def _():
        pltpu.sync_copy(shf, ph_v)
        acc = ph_v[0, pl.ds(0, _L)]
        for t in range(1, _NT):
            acc = acc + ph_v[t, pl.ds(0, _L)]
        tot_sumsq = jnp.sum(jnp.where(lane == 0, acc, jnp.float32(0.0)))
        t_val = jnp.max(lax.bitcast_convert_type(
            jnp.full((_L,), t_bits, jnp.int32), jnp.float32))
        kf = jnp.float32(_K)
        cvar = (s_gt + (kf - cnt_gt) * t_val) * jnp.float32(1.0 / _K)
        mse = tot_sumsq * jnp.float32(1.0 / _N)
        res = (1.0 - _LAMBDA) * mse + _LAMBDA * cvar
        out_stage[0, pl.ds(0, _L)] = jnp.full((_L,), res, jnp.float32)
        pltpu.sync_copy(out_stage, out_hbm)


_sc_call = functools.partial(
    pl.kernel,
    mesh=plsc.VectorSubcoreMesh(core_axis_name="c", subcore_axis_name="s"),
    out_type=jax.ShapeDtypeStruct((1, _L), jnp.float32),
    compiler_params=pltpu.CompilerParams(needs_layout_passes=False),
    scratch_types=[
        pltpu.VMEM((_NE,), jnp.float32),
        pltpu.VMEM((_SUB,), jnp.float32),
        pltpu.VMEM((_SUB,), jnp.float32),
        pltpu.VMEM((_NT * _HS,), jnp.int32),
        pltpu.VMEM((_NT * _HS,), jnp.float32),
        pltpu.VMEM((256,), jnp.int32),
        pltpu.VMEM((256,), jnp.float32),
        pltpu.VMEM((_NT, 256), jnp.int32),
        pltpu.VMEM((_NT, 256), jnp.float32),
        pltpu.VMEM((256,), jnp.int32),
        pltpu.VMEM((256,), jnp.float32),
        pltpu.VMEM((256,), jnp.float32),
        pltpu.VMEM((_NT, 256), jnp.float32),
        pltpu.VMEM((1, _L), jnp.float32),
        pltpu.VMEM_SHARED((_NT, 256), jnp.int32),
        pltpu.VMEM_SHARED((_NT, 256), jnp.int32),
        pltpu.VMEM_SHARED((_NT, 256), jnp.int32),
        pltpu.VMEM_SHARED((_NT, 256), jnp.int32),
        pltpu.VMEM_SHARED((_NT, 256), jnp.float32),
        pltpu.VMEM_SHARED((_NT, 256), jnp.float32),
        pltpu.VMEM_SHARED((_NT, 256), jnp.float32),
        pltpu.VMEM_SHARED((_NT, 256), jnp.float32),
        pltpu.VMEM_SHARED((_NT, 256), jnp.float32),
    ],
)(_sc_body)


@jax.jit
def kernel(pred, target):
    out = _sc_call(pred, target)
    return out[0, 0]


# speedup vs baseline: 1.5843x; 1.0780x over previous
import functools

import jax
import jax.numpy as jnp
from jax import lax
from jax.experimental import pallas as pl
from jax.experimental.pallas import tpu as pltpu
from jax.experimental.pallas import tpu_sc as plsc

_ALPHA = 0.95
_LAMBDA = 0.5
_N = 1048576
_K = int(_N * _ALPHA)
_NT = 16
_NE = _N // _NT
_SUB = 8192
_NSUB = _NE // _SUB
_L = 16
_HS = 257


def _sc_body(pred_hbm, tgt_hbm, out_hbm,
             err_v, stg_p, stg_t, stg_p1, stg_t1, sem_p0, sem_t0,
             sem_p1, sem_t1, hist_v, vhist_v, fold_v, vfold_v,
             gh_v, vgh_v, g_v, gv_v, partial_v, ph_v, out_stage,
             sh1, sh2, sh3, sh4, vsh1, vsh2, vsh3, vsh4, shf):
    sid = lax.axis_index("s")
    cid = lax.axis_index("c")
    base = sid * _NE
    lane = lax.iota(jnp.int32, _L)
    ones_i = jnp.ones((_L,), jnp.int32)

    def zero_hists(j, c):
        hist_v[pl.ds(j * _L, _L)] = jnp.zeros((_L,), jnp.int32)
        vhist_v[pl.ds(j * _L, _L)] = jnp.zeros((_L,), jnp.float32)
        return c

    lax.fori_loop(0, _NT * _HS // _L, zero_hists, 0, unroll=8)

    bufs = [(stg_p, stg_t, sem_p0, sem_t0), (stg_p1, stg_t1, sem_p1, sem_t1)]

    def start_stage(c):
        off = base + c * _SUB
        sp, st, s1, s2 = bufs[c % 2]
        hp = pltpu.async_copy(pred_hbm.at[pl.ds(off, _SUB)], sp, s1)
        ht = pltpu.async_copy(tgt_hbm.at[pl.ds(off, _SUB)], st, s2)
        return hp, ht

    handles = start_stage(0)
    acc_sq = jnp.zeros((_L,), jnp.float32)
    for c in range(_NSUB):
        nxt = start_stage(c + 1) if c + 1 < _NSUB else None
        handles[0].wait()
        handles[1].wait()
        sp, st = bufs[c % 2][0], bufs[c % 2][1]

        def inner(i, a, sp=sp, st=st, c=c):
            p = sp[pl.ds(i * _L, _L)]
            t = st[pl.ds(i * _L, _L)]
            e = jnp.abs(p - t)
            err_v[pl.ds(c * _SUB + i * _L, _L)] = e
            bits = lax.bitcast_convert_type(e, jnp.int32)
            idx = lane * _HS + lax.shift_right_logical(bits, 23)
            plsc.addupdate_scatter(hist_v, [idx], ones_i)
            plsc.addupdate_scatter(vhist_v, [idx], e)
            return a + e * e

        acc_sq = lax.fori_loop(0, _SUB // _L, inner, acc_sq, unroll=16)
        handles = nxt
    sum_sq = jnp.sum(acc_sq)

    def fold_publish(sh, vsh):
        def fold(j, c):
            acc = hist_v[pl.ds(j * _L, _L)]
            vacc = vhist_v[pl.ds(j * _L, _L)]
            for l in range(1, _NT):
                acc = acc + hist_v[pl.ds(l * _HS + j * _L, _L)]
                vacc = vacc + vhist_v[pl.ds(l * _HS + j * _L, _L)]
            fold_v[pl.ds(j * _L, _L)] = acc
            vfold_v[pl.ds(j * _L, _L)] = vacc
            return c

        lax.fori_loop(0, 16, fold, 0)
        pltpu.sync_copy(fold_v, sh.at[sid])
        pltpu.sync_copy(vfold_v, vsh.at[sid])
        plsc.subcore_barrier()

    def global_scan(sh, vsh, k_rem):
        pltpu.sync_copy(sh, gh_v)
        pltpu.sync_copy(vsh, vgh_v)

        def foldg(j, c):
            acc = gh_v[0, pl.ds(j * _L, _L)]
            vacc = vgh_v[0, pl.ds(j * _L, _L)]
            for t in range(1, _NT):
                acc = acc + gh_v[t, pl.ds(j * _L, _L)]
                vacc = vacc + vgh_v[t, pl.ds(j * _L, _L)]
            g_v[pl.ds(j * _L, _L)] = acc
            gv_v[pl.ds(j * _L, _L)] = vacc
            return c

        lax.fori_loop(0, 16, foldg, 0)

        def sufloop(jj, carry):
            best, run = carry
            j = 15 - jj
            g = g_v[pl.ds(j * _L, _L)]
            tot = jnp.sum(g)
            cum = plsc.cumsum(g)
            s = run + (tot - cum) + g
            ids = j * _L + lane
            cand = jnp.where(s >= k_rem, ids, jnp.int32(-1))
            return jnp.maximum(best, jnp.max(cand)), run + tot

        best, _ = lax.fori_loop(0, 16, sufloop,
                                (jnp.int32(-1), jnp.int32(0)))

        def aboveloop(j, carry):
            na, va = carry
            g = g_v[pl.ds(j * _L, _L)]
            gv = gv_v[pl.ds(j * _L, _L)]
            ids = j * _L + lane
            gtm = ids > best
            na = na + jnp.sum(jnp.where(gtm, g, jnp.int32(0)))
            va = va + jnp.sum(jnp.where(gtm, gv, jnp.float32(0.0)))
            return na, va

        n_above, v_above = lax.fori_loop(
            0, 16, aboveloop, (jnp.int32(0), jnp.float32(0.0)))
        return best, n_above, v_above

    fold_publish(sh1, vsh1)
    c1, na1, va1 = global_scan(sh1, vsh1, jnp.int32(_K))
    k_rem = jnp.int32(_K) - na1

    lax.fori_loop(0, _NT * _HS // _L, zero_hists, 0, unroll=8)

    def l2loop(i, off):
        e = err_v[pl.ds(i * _L, _L)]
        bits = lax.bitcast_convert_type(e, jnp.int32)
        m = lax.shift_right_logical(bits, 23) == c1
        idx = lane * _HS + (lax.shift_right_logical(bits, 15) & 0xFF)
        plsc.addupdate_scatter(hist_v, [idx], ones_i, mask=m)
        plsc.addupdate_scatter(vhist_v, [idx], e, mask=m)
        mcnt = plsc.cumsum(jnp.where(m, jnp.int32(1), jnp.int32(0)))
        plsc.store_scatter(err_v, [off + mcnt - 1], e, mask=m)
        return off + plsc.all_reduce_population_count(m)

    off2 = lax.fori_loop(0, _NE // _L, l2loop,
                         jnp.zeros((_L,), jnp.int32), unroll=16)
    n2 = jnp.max(off2)
    fold_publish(sh2, vsh2)
    c2, na2, va2 = global_scan(sh2, vsh2, k_rem)
    k_rem = k_rem - na2
    p2 = (c1 << 8) | c2

    lax.fori_loop(0, _NT * _HS // _L, zero_hists, 0, unroll=8)

    def l3loop(i, off):
        e = err_v[pl.ds(i * _L, _L)]
        bits = lax.bitcast_convert_type(e, jnp.int32)
        valid = (i * _L + lane) < n2
        m = jnp.logical_and(lax.shift_right_logical(bits, 15) == p2, valid)
        idx = lane * _HS + (lax.shift_right_logical(bits, 7) & 0xFF)
        plsc.addupdate_scatter(hist_v, [idx], ones_i, mask=m)
        plsc.addupdate_scatter(vhist_v, [idx], e, mask=m)
        mcnt = plsc.cumsum(jnp.where(m, jnp.int32(1), jnp.int32(0)))
        plsc.store_scatter(err_v, [off + mcnt - 1], e, mask=m)
        return off + plsc.all_reduce_population_count(m)

    off3 = lax.fori_loop(0, lax.shift_right_logical(n2 + _L - 1, 4), l3loop,
                         jnp.zeros((_L,), jnp.int32))
    n3 = jnp.max(off3)
    fold_publish(sh3, vsh3)
    c3, na3, va3 = global_scan(sh3, vsh3, k_rem)
    k_rem = k_rem - na3
    p3 = (p2 << 8) | c3

    lax.fori_loop(0, _NT * _HS // _L, zero_hists, 0, unroll=8)

    def l4loop(i, c):
        e = err_v[pl.ds(i * _L, _L)]
        bits = lax.bitcast_convert_type(e, jnp.int32)
        valid = (i * _L + lane) < n3
        m = jnp.logical_and(lax.shift_right_logical(bits, 7) == p3, valid)
        idx = lane * _HS + (bits & 0x7F)
        plsc.addupdate_scatter(hist_v, [idx], ones_i, mask=m)
        plsc.addupdate_scatter(vhist_v, [idx], e, mask=m)
        return c

    lax.fori_loop(0, lax.shift_right_logical(n3 + _L - 1, 4), l4loop, 0)
    fold_publish(sh4, vsh4)
    c4, na4, va4 = global_scan(sh4, vsh4, k_rem)
    t_bits = (p3 << 7) | c4

    cnt_gt = (na1 + na2 + na3 + na4).astype(jnp.float32)
    s_gt = va1 + va2 + va3 + va4

    def zero_partial(j, c):
        partial_v[pl.ds(j * _L, _L)] = jnp.zeros((_L,), jnp.float32)
        return c

    lax.fori_loop(0, 256 // _L, zero_partial, 0)
    partial_v[pl.ds(0, _L)] = jnp.where(lane == 0, sum_sq, jnp.float32(0.0))
    pltpu.sync_copy(partial_v, shf.at[sid])
    plsc.subcore_barrier()

    @pl.when(jnp.logical_and(sid == 0, cid == 0))
    def _():
        pltpu.sync_copy(shf, ph_v)
        acc = ph_v[0, pl.ds(0, _L)]
        for t in range(1, _NT):
            acc = acc + ph_v[t, pl.ds(0, _L)]
        tot_sumsq = jnp.sum(jnp.where(lane == 0, acc, jnp.float32(0.0)))
        t_val = jnp.max(lax.bitcast_convert_type(
            jnp.full((_L,), t_bits, jnp.int32), jnp.float32))
        kf = jnp.float32(_K)
        cvar = (s_gt + (kf - cnt_gt) * t_val) * jnp.float32(1.0 / _K)
        mse = tot_sumsq * jnp.float32(1.0 / _N)
        res = (1.0 - _LAMBDA) * mse + _LAMBDA * cvar
        out_stage[0, pl.ds(0, _L)] = jnp.full((_L,), res, jnp.float32)
        pltpu.sync_copy(out_stage, out_hbm)


_sc_call = functools.partial(
    pl.kernel,
    mesh=plsc.VectorSubcoreMesh(core_axis_name="c", subcore_axis_name="s"),
    out_type=jax.ShapeDtypeStruct((1, _L), jnp.float32),
    compiler_params=pltpu.CompilerParams(needs_layout_passes=False),
    scratch_types=[
        pltpu.VMEM((_NE,), jnp.float32),
        pltpu.VMEM((_SUB,), jnp.float32),
        pltpu.VMEM((_SUB,), jnp.float32),
        pltpu.VMEM((_SUB,), jnp.float32),
        pltpu.VMEM((_SUB,), jnp.float32),
        pltpu.SemaphoreType.DMA,
        pltpu.SemaphoreType.DMA,
        pltpu.SemaphoreType.DMA,
        pltpu.SemaphoreType.DMA,
        pltpu.VMEM((_NT * _HS,), jnp.int32),
        pltpu.VMEM((_NT * _HS,), jnp.float32),
        pltpu.VMEM((256,), jnp.int32),
        pltpu.VMEM((256,), jnp.float32),
        pltpu.VMEM((_NT, 256), jnp.int32),
        pltpu.VMEM((_NT, 256), jnp.float32),
        pltpu.VMEM((256,), jnp.int32),
        pltpu.VMEM((256,), jnp.float32),
        pltpu.VMEM((256,), jnp.float32),
        pltpu.VMEM((_NT, 256), jnp.float32),
        pltpu.VMEM((1, _L), jnp.float32),
        pltpu.VMEM_SHARED((_NT, 256), jnp.int32),
        pltpu.VMEM_SHARED((_NT, 256), jnp.int32),
        pltpu.VMEM_SHARED((_NT, 256), jnp.int32),
        pltpu.VMEM_SHARED((_NT, 256), jnp.int32),
        pltpu.VMEM_SHARED((_NT, 256), jnp.float32),
        pltpu.VMEM_SHARED((_NT, 256), jnp.float32),
        pltpu.VMEM_SHARED((_NT, 256), jnp.float32),
        pltpu.VMEM_SHARED((_NT, 256), jnp.float32),
        pltpu.VMEM_SHARED((_NT, 256), jnp.float32),
    ],
)(_sc_body)


@jax.jit
def kernel(pred, target):
    out = _sc_call(pred, target)
    return out[0, 0]


# speedup vs baseline: 1.5876x; 1.0021x over previous
import functools

import jax
import jax.numpy as jnp
from jax import lax
from jax.experimental import pallas as pl
from jax.experimental.pallas import tpu as pltpu
from jax.experimental.pallas import tpu_sc as plsc

_ALPHA = 0.95
_LAMBDA = 0.5
_N = 1048576
_K = int(_N * _ALPHA)
_NT = 16
_NE = _N // _NT
_SUB = 8192
_NSUB = _NE // _SUB
_L = 16
_HS = 257


def _sc_body(pred_hbm, tgt_hbm, out_hbm,
             err_v, stg_p, stg_t, stg_p1, stg_t1, sem_p0, sem_t0,
             sem_p1, sem_t1, hist_v, vhist_v, fold_v, vfold_v,
             gh_v, vgh_v, g_v, gv_v, partial_v, ph_v, out_stage,
             sh1, sh2, sh3, sh4, vsh1, vsh2, vsh3, vsh4, shf):
    sid = lax.axis_index("s")
    cid = lax.axis_index("c")
    base = sid * _NE
    lane = lax.iota(jnp.int32, _L)
    ones_i = jnp.ones((_L,), jnp.int32)

    def zero_hists(j, c):
        hist_v[pl.ds(j * _L, _L)] = jnp.zeros((_L,), jnp.int32)
        vhist_v[pl.ds(j * _L, _L)] = jnp.zeros((_L,), jnp.float32)
        return c

    lax.fori_loop(0, _NT * _HS // _L, zero_hists, 0, unroll=8)

    bufs = [(stg_p, stg_t, sem_p0, sem_t0), (stg_p1, stg_t1, sem_p1, sem_t1)]

    def start_stage(c):
        off = base + c * _SUB
        sp, st, s1, s2 = bufs[c % 2]
        hp = pltpu.async_copy(pred_hbm.at[pl.ds(off, _SUB)], sp, s1)
        ht = pltpu.async_copy(tgt_hbm.at[pl.ds(off, _SUB)], st, s2)
        return hp, ht

    handles = start_stage(0)
    _z = jnp.zeros((_L,), jnp.float32)
    acc_sq = (_z, _z, _z, _z)
    for c in range(_NSUB):
        nxt = start_stage(c + 1) if c + 1 < _NSUB else None
        handles[0].wait()
        handles[1].wait()
        sp, st = bufs[c % 2][0], bufs[c % 2][1]

        def inner(i, accs, sp=sp, st=st, c=c):
            sq = []
            for j in range(4):
                p = sp[pl.ds((i * 4 + j) * _L, _L)]
                t = st[pl.ds((i * 4 + j) * _L, _L)]
                e = jnp.abs(p - t)
                err_v[pl.ds(c * _SUB + (i * 4 + j) * _L, _L)] = e
                bits = lax.bitcast_convert_type(e, jnp.int32)
                idx = lane * _HS + lax.shift_right_logical(bits, 23)
                plsc.addupdate_scatter(hist_v, [idx], ones_i)
                plsc.addupdate_scatter(vhist_v, [idx], e)
                sq.append(e * e)
            return tuple(a + s for a, s in zip(accs, sq))

        acc_sq = lax.fori_loop(0, _SUB // _L // 4, inner, acc_sq, unroll=4)
        handles = nxt
    sum_sq = jnp.sum(acc_sq[0] + acc_sq[1] + acc_sq[2] + acc_sq[3])

    def fold_publish(sh, vsh):
        def fold(j, c):
            acc = hist_v[pl.ds(j * _L, _L)]
            vacc = vhist_v[pl.ds(j * _L, _L)]
            for l in range(1, _NT):
                acc = acc + hist_v[pl.ds(l * _HS + j * _L, _L)]
                vacc = vacc + vhist_v[pl.ds(l * _HS + j * _L, _L)]
            fold_v[pl.ds(j * _L, _L)] = acc
            vfold_v[pl.ds(j * _L, _L)] = vacc
            return c

        lax.fori_loop(0, 16, fold, 0)
        pltpu.sync_copy(fold_v, sh.at[sid])
        pltpu.sync_copy(vfold_v, vsh.at[sid])
        plsc.subcore_barrier()

    def global_scan(sh, vsh, k_rem):
        pltpu.sync_copy(sh, gh_v)
        pltpu.sync_copy(vsh, vgh_v)

        def foldg(j, c):
            acc = gh_v[0, pl.ds(j * _L, _L)]
            vacc = vgh_v[0, pl.ds(j * _L, _L)]
            for t in range(1, _NT):
                acc = acc + gh_v[t, pl.ds(j * _L, _L)]
                vacc = vacc + vgh_v[t, pl.ds(j * _L, _L)]
            g_v[pl.ds(j * _L, _L)] = acc
            gv_v[pl.ds(j * _L, _L)] = vacc
            return c

        lax.fori_loop(0, 16, foldg, 0)

        def sufloop(jj, carry):
            best, run = carry
            j = 15 - jj
            g = g_v[pl.ds(j * _L, _L)]
            tot = jnp.sum(g)
            cum = plsc.cumsum(g)
            s = run + (tot - cum) + g
            ids = j * _L + lane
            cand = jnp.where(s >= k_rem, ids, jnp.int32(-1))
            return jnp.maximum(best, jnp.max(cand)), run + tot

        best, _ = lax.fori_loop(0, 16, sufloop,
                                (jnp.int32(-1), jnp.int32(0)))

        def aboveloop(j, carry):
            na, va = carry
            g = g_v[pl.ds(j * _L, _L)]
            gv = gv_v[pl.ds(j * _L, _L)]
            ids = j * _L + lane
            gtm = ids > best
            na = na + jnp.sum(jnp.where(gtm, g, jnp.int32(0)))
            va = va + jnp.sum(jnp.where(gtm, gv, jnp.float32(0.0)))
            return na, va

        n_above, v_above = lax.fori_loop(
            0, 16, aboveloop, (jnp.int32(0), jnp.float32(0.0)))
        return best, n_above, v_above

    fold_publish(sh1, vsh1)
    c1, na1, va1 = global_scan(sh1, vsh1, jnp.int32(_K))
    k_rem = jnp.int32(_K) - na1

    lax.fori_loop(0, _NT * _HS // _L, zero_hists, 0, unroll=8)

    def l2loop(i, off):
        e = err_v[pl.ds(i * _L, _L)]
        bits = lax.bitcast_convert_type(e, jnp.int32)
        m = lax.shift_right_logical(bits, 23) == c1
        idx = lane * _HS + (lax.shift_right_logical(bits, 15) & 0xFF)
        plsc.addupdate_scatter(hist_v, [idx], ones_i, mask=m)
        plsc.addupdate_scatter(vhist_v, [idx], e, mask=m)
        mcnt = plsc.cumsum(jnp.where(m, jnp.int32(1), jnp.int32(0)))
        plsc.store_scatter(err_v, [off + mcnt - 1], e, mask=m)
        return off + plsc.all_reduce_population_count(m)

    off2 = lax.fori_loop(0, _NE // _L, l2loop,
                         jnp.zeros((_L,), jnp.int32), unroll=16)
    n2 = jnp.max(off2)
    fold_publish(sh2, vsh2)
    c2, na2, va2 = global_scan(sh2, vsh2, k_rem)
    k_rem = k_rem - na2
    p2 = (c1 << 8) | c2

    lax.fori_loop(0, _NT * _HS // _L, zero_hists, 0, unroll=8)

    def l3loop(i, off):
        e = err_v[pl.ds(i * _L, _L)]
        bits = lax.bitcast_convert_type(e, jnp.int32)
        valid = (i * _L + lane) < n2
        m = jnp.logical_and(lax.shift_right_logical(bits, 15) == p2, valid)
        idx = lane * _HS + (lax.shift_right_logical(bits, 7) & 0xFF)
        plsc.addupdate_scatter(hist_v, [idx], ones_i, mask=m)
        plsc.addupdate_scatter(vhist_v, [idx], e, mask=m)
        mcnt = plsc.cumsum(jnp.where(m, jnp.int32(1), jnp.int32(0)))
        plsc.store_scatter(err_v, [off + mcnt - 1], e, mask=m)
        return off + plsc.all_reduce_population_count(m)

    off3 = lax.fori_loop(0, lax.shift_right_logical(n2 + _L - 1, 4), l3loop,
                         jnp.zeros((_L,), jnp.int32))
    n3 = jnp.max(off3)
    fold_publish(sh3, vsh3)
    c3, na3, va3 = global_scan(sh3, vsh3, k_rem)
    k_rem = k_rem - na3
    p3 = (p2 << 8) | c3

    lax.fori_loop(0, _NT * _HS // _L, zero_hists, 0, unroll=8)

    def l4loop(i, c):
        e = err_v[pl.ds(i * _L, _L)]
        bits = lax.bitcast_convert_type(e, jnp.int32)
        valid = (i * _L + lane) < n3
        m = jnp.logical_and(lax.shift_right_logical(bits, 7) == p3, valid)
        idx = lane * _HS + (bits & 0x7F)
        plsc.addupdate_scatter(hist_v, [idx], ones_i, mask=m)
        plsc.addupdate_scatter(vhist_v, [idx], e, mask=m)
        return c

    lax.fori_loop(0, lax.shift_right_logical(n3 + _L - 1, 4), l4loop, 0)
    fold_publish(sh4, vsh4)
    c4, na4, va4 = global_scan(sh4, vsh4, k_rem)
    t_bits = (p3 << 7) | c4

    cnt_gt = (na1 + na2 + na3 + na4).astype(jnp.float32)
    s_gt = va1 + va2 + va3 + va4

    def zero_partial(j, c):
        partial_v[pl.ds(j * _L, _L)] = jnp.zeros((_L,), jnp.float32)
        return c

    lax.fori_loop(0, 256 // _L, zero_partial, 0)
    partial_v[pl.ds(0, _L)] = jnp.where(lane == 0, sum_sq, jnp.float32(0.0))
    pltpu.sync_copy(partial_v, shf.at[sid])
    plsc.subcore_barrier()

    @pl.when(jnp.logical_and(sid == 0, cid == 0))
    def _():
        pltpu.sync_copy(shf, ph_v)
        acc = ph_v[0, pl.ds(0, _L)]
        for t in range(1, _NT):
            acc = acc + ph_v[t, pl.ds(0, _L)]
        tot_sumsq = jnp.sum(jnp.where(lane == 0, acc, jnp.float32(0.0)))
        t_val = jnp.max(lax.bitcast_convert_type(
            jnp.full((_L,), t_bits, jnp.int32), jnp.float32))
        kf = jnp.float32(_K)
        cvar = (s_gt + (kf - cnt_gt) * t_val) * jnp.float32(1.0 / _K)
        mse = tot_sumsq * jnp.float32(1.0 / _N)
        res = (1.0 - _LAMBDA) * mse + _LAMBDA * cvar
        out_stage[0, pl.ds(0, _L)] = jnp.full((_L,), res, jnp.float32)
        pltpu.sync_copy(out_stage, out_hbm)


_sc_call = functools.partial(
    pl.kernel,
    mesh=plsc.VectorSubcoreMesh(core_axis_name="c", subcore_axis_name="s"),
    out_type=jax.ShapeDtypeStruct((1, _L), jnp.float32),
    compiler_params=pltpu.CompilerParams(needs_layout_passes=False),
    scratch_types=[
        pltpu.VMEM((_NE,), jnp.float32),
        pltpu.VMEM((_SUB,), jnp.float32),
        pltpu.VMEM((_SUB,), jnp.float32),
        pltpu.VMEM((_SUB,), jnp.float32),
        pltpu.VMEM((_SUB,), jnp.float32),
        pltpu.SemaphoreType.DMA,
        pltpu.SemaphoreType.DMA,
        pltpu.SemaphoreType.DMA,
        pltpu.SemaphoreType.DMA,
        pltpu.VMEM((_NT * _HS,), jnp.int32),
        pltpu.VMEM((_NT * _HS,), jnp.float32),
        pltpu.VMEM((256,), jnp.int32),
        pltpu.VMEM((256,), jnp.float32),
        pltpu.VMEM((_NT, 256), jnp.int32),
        pltpu.VMEM((_NT, 256), jnp.float32),
        pltpu.VMEM((256,), jnp.int32),
        pltpu.VMEM((256,), jnp.float32),
        pltpu.VMEM((256,), jnp.float32),
        pltpu.VMEM((_NT, 256), jnp.float32),
        pltpu.VMEM((1, _L), jnp.float32),
        pltpu.VMEM_SHARED((_NT, 256), jnp.int32),
        pltpu.VMEM_SHARED((_NT, 256), jnp.int32),
        pltpu.VMEM_SHARED((_NT, 256), jnp.int32),
        pltpu.VMEM_SHARED((_NT, 256), jnp.int32),
        pltpu.VMEM_SHARED((_NT, 256), jnp.float32),
        pltpu.VMEM_SHARED((_NT, 256), jnp.float32),
        pltpu.VMEM_SHARED((_NT, 256), jnp.float32),
        pltpu.VMEM_SHARED((_NT, 256), jnp.float32),
        pltpu.VMEM_SHARED((_NT, 256), jnp.float32),
    ],
)(_sc_body)


@jax.jit
def kernel(pred, target):
    out = _sc_call(pred, target)
    return out[0, 0]
